# baseline (speedup 1.0000x reference)
import functools

import jax
import jax.numpy as jnp
from jax.experimental import pallas as pl

_HIDDEN = 4096
_TOKENS = 4096
_EXPERTS = 64
_BLOCK = 512


def _moe_router_kernel(x_ref, wt_ref, zeros_ref, logits_ref):
    zeros_ref[...] = jnp.zeros_like(zeros_ref)
    logits_ref[...] = jnp.dot(
        x_ref[...], wt_ref[...], preferred_element_type=jnp.float32
    )


@functools.partial(jax.jit, static_argnames=())
def kernel(hidden_states, w_gate):
    batch, seq, hidden = hidden_states.shape
    x = hidden_states.reshape(-1, hidden)
    tokens = x.shape[0]
    wt = w_gate.T
    experts = wt.shape[1]

    grid = (tokens // _BLOCK,)
    zeros2d, logits = pl.pallas_call(
        _moe_router_kernel,
        grid=grid,
        in_specs=[
            pl.BlockSpec((_BLOCK, hidden), lambda i: (i, 0)),
            pl.BlockSpec((hidden, experts), lambda i: (0, 0)),
        ],
        out_specs=[
            pl.BlockSpec((_BLOCK, hidden), lambda i: (i, 0)),
            pl.BlockSpec((_BLOCK, experts), lambda i: (i, 0)),
        ],
        out_shape=[
            jax.ShapeDtypeStruct((tokens, hidden), hidden_states.dtype),
            jax.ShapeDtypeStruct((tokens, experts), jnp.float32),
        ],
    )(x, wt)
    return zeros2d.reshape(batch, seq, hidden), logits
